# Initial kernel scaffold; baseline (speedup 1.0000x reference)
import jax
import jax.numpy as jnp
from jax.experimental import pallas as pl


def kernel(spot_emb, user_emb, user_spot, spot_edge_index, spot_edge_weight, user_edge_index):
    raise NotImplementedError("write your pallas kernel here")


# speedup vs baseline: 8.8526x; 8.8526x over previous
import functools

import jax
import jax.numpy as jnp
from jax import lax
from jax.experimental import pallas as pl
from jax.experimental.pallas import tpu as pltpu
from jax.experimental.pallas import tpu_sc as plsc

N_USER = 27094
M_SPOT = 42852
HID = 64
HH = HID // 2

NU = 28672
MS = 43008

E_US = 1000000
E_SPOT = 685632
E_USER = 541880
E_US_P = 16384 * 62
E_SP_P = 16384 * 42
E_UE_P = 16384 * 34

_MESH = plsc.VectorSubcoreMesh(core_axis_name="c", subcore_axis_name="s")


RU = NU // 16
RS = MS // 16


def _hist_body(u_ref, s_ref, ud_ref, sd_ref, w_ref,
               cu_ref, cs_ref, du_ref, ds_ref,
               h1, h2, st1, st2, wst, ridx, stage, a_cu, a_cs, a_du, a_ds):
    c = lax.axis_index("c")
    s = lax.axis_index("s")
    z16 = jnp.zeros((16,), jnp.float32)
    o16 = jnp.ones((16,), jnp.float32)
    i16 = jnp.arange(16, dtype=jnp.int32)

    @pl.loop(0, NU // 16)
    def _(i):
        h1[pl.ds(i * 16, 16)] = z16

    @pl.loop(0, MS // 16)
    def _(i):
        h2[pl.ds(i * 16, 16)] = z16

    @pl.loop(0, 128)
    def _(r):
        stage[r, pl.ds(0, 16)] = z16

    @pl.loop(0, RS // 128)
    def _(r):
        @pl.loop(0, 8)
        def _(g):
            ridx[r, pl.ds(g * 16, 16)] = i16 + (r * 128 + g * 16)

    @pl.when(s == 0)
    def _():
        for acc, nrow in ((a_cu, RU), (a_cs, RS), (a_du, RU), (a_ds, RS)):
            @pl.loop(0, nrow // 128)
            def _(k):
                pltpu.sync_copy(stage, acc.at[pl.ds(k * 128, 128)])

    plsc.subcore_barrier()

    t = c * 16 + s

    def count_phase(idx_ref, hist, ep, wref=None):
        et = ep // 32

        @pl.loop(0, et // 1024)
        def _(b):
            base = t * et + b * 1024
            pltpu.sync_copy(idx_ref.at[pl.ds(base, 1024)], st1)
            if wref is not None:
                pltpu.sync_copy(wref.at[pl.ds(base, 1024)], wst)

            @pl.loop(0, 64)
            def _(j):
                iv = st1[pl.ds(j * 16, 16)]
                vals = o16 if wref is None else wst[pl.ds(j * 16, 16)]
                plsc.addupdate_scatter(hist, [iv], vals)

    def reduce_phase(hist, acc, nrow):
        @pl.loop(0, nrow // 128)
        def _(k):
            @pl.loop(0, 128)
            def _(r):
                stage[r, pl.ds(0, 16)] = hist[pl.ds((k * 128 + r) * 16, 16)]
            pltpu.sync_copy(stage, acc.at[ridx.at[k]], add=True)

    def zero_hist(hist, n):
        @pl.loop(0, n // 16)
        def _(i):
            hist[pl.ds(i * 16, 16)] = z16

    eta = E_US_P // 32

    @pl.loop(0, eta // 1024)
    def _(b):
        base = t * eta + b * 1024
        pltpu.sync_copy(u_ref.at[pl.ds(base, 1024)], st1)
        pltpu.sync_copy(s_ref.at[pl.ds(base, 1024)], st2)

        @pl.loop(0, 64)
        def _(j):
            plsc.addupdate_scatter(h1, [st1[pl.ds(j * 16, 16)]], o16)
            plsc.addupdate_scatter(h2, [st2[pl.ds(j * 16, 16)]], o16)

    reduce_phase(h1, a_cu, RU)
    reduce_phase(h2, a_cs, RS)

    zero_hist(h1, NU)
    count_phase(ud_ref, h1, E_UE_P)
    reduce_phase(h1, a_du, RU)

    zero_hist(h2, MS)
    count_phase(sd_ref, h2, E_SP_P, w_ref)
    reduce_phase(h2, a_ds, RS)

    plsc.subcore_barrier()

    ru = RU // 16
    rs = RS // 16
    pltpu.sync_copy(a_cu.at[pl.ds(s * ru, ru)], cu_ref.at[c, pl.ds(s * ru, ru)])
    pltpu.sync_copy(a_cs.at[pl.ds(s * rs, rs)], cs_ref.at[c, pl.ds(s * rs, rs)])
    pltpu.sync_copy(a_du.at[pl.ds(s * ru, ru)], du_ref.at[c, pl.ds(s * ru, ru)])
    pltpu.sync_copy(a_ds.at[pl.ds(s * rs, rs)], ds_ref.at[c, pl.ds(s * rs, rs)])


_hist_call = pl.kernel(
    _hist_body,
    out_type=(
        jax.ShapeDtypeStruct((2, RU, 16), jnp.float32),
        jax.ShapeDtypeStruct((2, RS, 16), jnp.float32),
        jax.ShapeDtypeStruct((2, RU, 16), jnp.float32),
        jax.ShapeDtypeStruct((2, RS, 16), jnp.float32),
    ),
    mesh=_MESH,
    compiler_params=pltpu.CompilerParams(needs_layout_passes=False, use_tc_tiling_on_sc=False),
    scratch_types=[
        pltpu.VMEM((NU,), jnp.float32),
        pltpu.VMEM((MS,), jnp.float32),
        pltpu.VMEM((1024,), jnp.int32),
        pltpu.VMEM((1024,), jnp.int32),
        pltpu.VMEM((1024,), jnp.float32),
        pltpu.VMEM((RS // 128, 128), jnp.int32),
        pltpu.VMEM((128, 16), jnp.float32),
        pltpu.VMEM_SHARED((RU, 16), jnp.float32),
        pltpu.VMEM_SHARED((RS, 16), jnp.float32),
        pltpu.VMEM_SHARED((RU, 16), jnp.float32),
        pltpu.VMEM_SHARED((RS, 16), jnp.float32),
    ],
)


def _make_spmv(weighted, EP, VSP, VDP):
    def body(*refs):
        if weighted:
            (src_ref, dst_ref, tbl_ref, w_ref, out_ref,
             acc, src_st, dst_st, w_st, rows, zer, sem) = refs
        else:
            (src_ref, dst_ref, tbl_ref, out_ref,
             acc, src_st, dst_st, w_st, rows, zer, sem) = refs
        c = lax.axis_index("c")
        s = lax.axis_index("s")
        z16 = jnp.zeros((16,), jnp.float32)

        @pl.loop(0, 128)
        def _(r):
            zer[r, pl.ds(0, 16)] = z16
            zer[r, pl.ds(16, 16)] = z16

        R = VDP // 16

        @pl.loop(0, R // 128)
        def _(k):
            pltpu.sync_copy(zer, acc.at[pl.ds(s * R + k * 128, 128)])

        plsc.subcore_barrier()

        ET = EP // 16
        ETR = ET // 128
        off = c * VSP

        @pl.loop(0, ET // 1024)
        def _(b):
            base = s * ET + b * 1024
            row0 = s * ETR + b * 8
            pltpu.sync_copy(src_ref.at[pl.ds(row0, 8)], src_st)
            pltpu.sync_copy(dst_ref.at[pl.ds(row0, 8)], dst_st)
            if weighted:
                pltpu.sync_copy(w_ref.at[pl.ds(base, 1024)], w_st)

            @pl.loop(0, 8)
            def _(r):
                @pl.loop(0, 8)
                def _(g):
                    v = src_st[r, pl.ds(g * 16, 16)]
                    src_st[r, pl.ds(g * 16, 16)] = v + off

            @pl.loop(0, 8)
            def _(k):
                pltpu.async_copy(tbl_ref.at[src_st.at[k]], rows, sem).wait()
                if weighted:
                    @pl.loop(0, 8)
                    def _(g):
                        wv = w_st[pl.ds(k * 128 + g * 16, 16)]
                        for j in range(16):
                            e = g * 16 + j
                            wsc = wv[j]
                            rows[e, pl.ds(0, 16)] = rows[e, pl.ds(0, 16)] * wsc
                            rows[e, pl.ds(16, 16)] = (
                                rows[e, pl.ds(16, 16)] * wsc)
                pltpu.sync_copy(rows, acc.at[dst_st.at[k]], add=True)

        plsc.subcore_barrier()

        @pl.loop(0, R // 128)
        def _(k):
            r0 = s * R + k * 128
            pltpu.sync_copy(acc.at[pl.ds(r0, 128)],
                            out_ref.at[pl.ds(c * VDP + r0, 128)])

    return pl.kernel(
        body,
        out_type=jax.ShapeDtypeStruct((2 * VDP, HH), jnp.float32),
        mesh=_MESH,
        compiler_params=pltpu.CompilerParams(use_tc_tiling_on_sc=False),
        scratch_types=[
            pltpu.VMEM_SHARED((VDP, HH), jnp.float32),
            pltpu.VMEM((8, 128), jnp.int32),
            pltpu.VMEM((8, 128), jnp.int32),
            pltpu.VMEM((1024,), jnp.float32),
            pltpu.VMEM((128, HH), jnp.float32),
            pltpu.VMEM((128, HH), jnp.float32),
            pltpu.SemaphoreType.DMA,
        ],
    )


def _pad1(x, n, val):
    return jnp.pad(x, (0, n - x.shape[0]), constant_values=val)


def _stack_halves(x, vp):
    xp = jnp.pad(x, ((0, vp - x.shape[0]), (0, 0)))
    return jnp.concatenate([xp[:, :HH], xp[:, HH:]], axis=0)


def _unstack(xs, vr, vp):
    return jnp.concatenate([xs[:vr], xs[vp:vp + vr]], axis=1)


_spmv_uu = _make_spmv(False, E_UE_P, NU, NU)
_spmv_ss = _make_spmv(True, E_SP_P, MS, MS)
_spmv_su = _make_spmv(False, E_US_P, MS, NU)
_spmv_us = _make_spmv(False, E_US_P, NU, MS)


def kernel(spot_emb, user_emb, user_spot, spot_edge_index, spot_edge_weight,
           user_edge_index):
    u = _pad1(user_spot[0], E_US_P, N_USER)
    sp = _pad1(user_spot[1], E_US_P, M_SPOT)
    ue_s = _pad1(user_edge_index[0], E_UE_P, N_USER)
    ue_d = _pad1(user_edge_index[1], E_UE_P, N_USER)
    se_s = _pad1(spot_edge_index[0], E_SP_P, M_SPOT)
    se_d = _pad1(spot_edge_index[1], E_SP_P, M_SPOT)
    w = _pad1(spot_edge_weight, E_SP_P, 0.0)

    cu2, cs2, du2, ds2 = _hist_call(u, sp, ue_d, se_d, w)
    cnt_u = (cu2[0] + cu2[1]).reshape(NU)
    cnt_s = (cs2[0] + cs2[1]).reshape(MS)
    deg_u = (du2[0] + du2[1]).reshape(NU)
    deg_s = (ds2[0] + ds2[1]).reshape(MS)
    uinv = jnp.where(cnt_u > 0, lax.rsqrt(jnp.maximum(cnt_u, 1e-30)), 0.0)
    sinv = jnp.where(cnt_s > 0, lax.rsqrt(jnp.maximum(cnt_s, 1e-30)), 0.0)
    dui = jnp.where(deg_u > 0, lax.rsqrt(jnp.maximum(deg_u, 1e-30)), 0.0)
    dsi = jnp.where(deg_s > 0, lax.rsqrt(jnp.maximum(deg_s, 1e-30)), 0.0)
    uinv2 = jnp.concatenate([uinv, uinv])[:, None]
    sinv2 = jnp.concatenate([sinv, sinv])[:, None]
    dui2 = jnp.concatenate([dui, dui])[:, None]
    dsi2 = jnp.concatenate([dsi, dsi])[:, None]

    u2d = u.reshape(E_US_P // 128, 128)
    s2d = sp.reshape(E_US_P // 128, 128)
    ued2d = ue_d.reshape(E_UE_P // 128, 128)
    ues2d = ue_s.reshape(E_UE_P // 128, 128)
    sed2d = se_d.reshape(E_SP_P // 128, 128)
    ses2d = se_s.reshape(E_SP_P // 128, 128)

    user_x = _stack_halves(user_emb, NU)
    spot_x = _stack_halves(spot_emb, MS)

    cat = _spmv_uu(ues2d, ued2d, user_x * dui2)
    user_x = user_x + cat * dui2
    cat = _spmv_ss(ses2d, sed2d, spot_x * dsi2, w)
    spot_x = spot_x + cat * dsi2

    u_out = user_x
    s_out = spot_x
    for _ in range(3):
        nu_ = uinv2 * _spmv_su(s2d, u2d, spot_x * sinv2)
        ns_ = sinv2 * _spmv_us(u2d, s2d, user_x * uinv2)
        user_x = nu_
        spot_x = ns_
        u_out = u_out + user_x
        s_out = s_out + spot_x

    s_out = s_out * 0.25
    u_out = u_out * 0.25
    return _unstack(s_out, M_SPOT, MS), _unstack(u_out, N_USER, NU)


# speedup vs baseline: 9.9074x; 1.1192x over previous
import functools

import jax
import jax.numpy as jnp
from jax import lax
from jax.experimental import pallas as pl
from jax.experimental.pallas import tpu as pltpu
from jax.experimental.pallas import tpu_sc as plsc

N_USER = 27094
M_SPOT = 42852
HID = 64
HH = HID // 2

NU = 28672
MS = 43008

E_US = 1000000
E_SPOT = 685632
E_USER = 541880
E_US_P = 16384 * 62
E_SP_P = 16384 * 42
E_UE_P = 16384 * 34

_MESH = plsc.VectorSubcoreMesh(core_axis_name="c", subcore_axis_name="s")


RU = NU // 16
RS = MS // 16


def _hist_body(u_ref, s_ref, ud_ref, sd_ref, w_ref,
               cu_ref, cs_ref, du_ref, ds_ref,
               h1, h2, st1, st2, wst, ridx, stage, a_cu, a_cs, a_du, a_ds):
    c = lax.axis_index("c")
    s = lax.axis_index("s")
    z16 = jnp.zeros((16,), jnp.float32)
    o16 = jnp.ones((16,), jnp.float32)
    i16 = jnp.arange(16, dtype=jnp.int32)

    @pl.loop(0, NU // 16)
    def _(i):
        h1[pl.ds(i * 16, 16)] = z16

    @pl.loop(0, MS // 16)
    def _(i):
        h2[pl.ds(i * 16, 16)] = z16

    @pl.loop(0, 128)
    def _(r):
        stage[r, pl.ds(0, 16)] = z16

    @pl.loop(0, RS // 128)
    def _(r):
        @pl.loop(0, 8)
        def _(g):
            ridx[r, pl.ds(g * 16, 16)] = i16 + (r * 128 + g * 16)

    @pl.when(s == 0)
    def _():
        for acc, nrow in ((a_cu, RU), (a_cs, RS), (a_du, RU), (a_ds, RS)):
            @pl.loop(0, nrow // 128)
            def _(k):
                pltpu.sync_copy(stage, acc.at[pl.ds(k * 128, 128)])

    plsc.subcore_barrier()

    t = c * 16 + s

    def count_phase(idx_ref, hist, ep, wref=None):
        et = ep // 32

        @pl.loop(0, et // 1024)
        def _(b):
            base = t * et + b * 1024
            pltpu.sync_copy(idx_ref.at[pl.ds(base, 1024)], st1)
            if wref is not None:
                pltpu.sync_copy(wref.at[pl.ds(base, 1024)], wst)

            @pl.loop(0, 64)
            def _(j):
                iv = st1[pl.ds(j * 16, 16)]
                vals = o16 if wref is None else wst[pl.ds(j * 16, 16)]
                plsc.addupdate_scatter(hist, [iv], vals)

    def reduce_phase(hist, acc, nrow):
        @pl.loop(0, nrow // 128)
        def _(k):
            @pl.loop(0, 128)
            def _(r):
                stage[r, pl.ds(0, 16)] = hist[pl.ds((k * 128 + r) * 16, 16)]
            pltpu.sync_copy(stage, acc.at[ridx.at[k]], add=True)

    def zero_hist(hist, n):
        @pl.loop(0, n // 16)
        def _(i):
            hist[pl.ds(i * 16, 16)] = z16

    eta = E_US_P // 32

    @pl.loop(0, eta // 1024)
    def _(b):
        base = t * eta + b * 1024
        pltpu.sync_copy(u_ref.at[pl.ds(base, 1024)], st1)
        pltpu.sync_copy(s_ref.at[pl.ds(base, 1024)], st2)

        @pl.loop(0, 64)
        def _(j):
            plsc.addupdate_scatter(h1, [st1[pl.ds(j * 16, 16)]], o16)
            plsc.addupdate_scatter(h2, [st2[pl.ds(j * 16, 16)]], o16)

    reduce_phase(h1, a_cu, RU)
    reduce_phase(h2, a_cs, RS)

    zero_hist(h1, NU)
    count_phase(ud_ref, h1, E_UE_P)
    reduce_phase(h1, a_du, RU)

    zero_hist(h2, MS)
    count_phase(sd_ref, h2, E_SP_P, w_ref)
    reduce_phase(h2, a_ds, RS)

    plsc.subcore_barrier()

    ru = RU // 16
    rs = RS // 16
    pltpu.sync_copy(a_cu.at[pl.ds(s * ru, ru)], cu_ref.at[c, pl.ds(s * ru, ru)])
    pltpu.sync_copy(a_cs.at[pl.ds(s * rs, rs)], cs_ref.at[c, pl.ds(s * rs, rs)])
    pltpu.sync_copy(a_du.at[pl.ds(s * ru, ru)], du_ref.at[c, pl.ds(s * ru, ru)])
    pltpu.sync_copy(a_ds.at[pl.ds(s * rs, rs)], ds_ref.at[c, pl.ds(s * rs, rs)])


_hist_call = pl.kernel(
    _hist_body,
    out_type=(
        jax.ShapeDtypeStruct((2, RU, 16), jnp.float32),
        jax.ShapeDtypeStruct((2, RS, 16), jnp.float32),
        jax.ShapeDtypeStruct((2, RU, 16), jnp.float32),
        jax.ShapeDtypeStruct((2, RS, 16), jnp.float32),
    ),
    mesh=_MESH,
    compiler_params=pltpu.CompilerParams(needs_layout_passes=False, use_tc_tiling_on_sc=False),
    scratch_types=[
        pltpu.VMEM((NU,), jnp.float32),
        pltpu.VMEM((MS,), jnp.float32),
        pltpu.VMEM((1024,), jnp.int32),
        pltpu.VMEM((1024,), jnp.int32),
        pltpu.VMEM((1024,), jnp.float32),
        pltpu.VMEM((RS // 128, 128), jnp.int32),
        pltpu.VMEM((128, 16), jnp.float32),
        pltpu.VMEM_SHARED((RU, 16), jnp.float32),
        pltpu.VMEM_SHARED((RS, 16), jnp.float32),
        pltpu.VMEM_SHARED((RU, 16), jnp.float32),
        pltpu.VMEM_SHARED((RS, 16), jnp.float32),
    ],
)


def _make_spmv(weighted, EP, VSP, VDP):
    def body(*refs):
        if weighted:
            (src_ref, dst_ref, tbl_ref, w_ref, out_ref,
             acc, src_st, dst_st, w_st, rows0, rows1, zer,
             sem0, sem1) = refs
        else:
            (src_ref, dst_ref, tbl_ref, out_ref,
             acc, src_st, dst_st, w_st, rows0, rows1, zer,
             sem0, sem1) = refs
        c = lax.axis_index("c")
        s = lax.axis_index("s")
        z16 = jnp.zeros((16,), jnp.float32)

        @pl.loop(0, 128)
        def _(r):
            zer[r, pl.ds(0, 16)] = z16
            zer[r, pl.ds(16, 16)] = z16

        R = VDP // 16

        @pl.loop(0, R // 128)
        def _(k):
            pltpu.sync_copy(zer, acc.at[pl.ds(s * R + k * 128, 128)])

        plsc.subcore_barrier()

        ET = EP // 16
        ETR = ET // 128
        off = c * VSP

        @pl.loop(0, ET // 1024)
        def _(b):
            base = s * ET + b * 1024
            row0 = s * ETR + b * 8
            pltpu.sync_copy(src_ref.at[pl.ds(row0, 8)], src_st)
            pltpu.sync_copy(dst_ref.at[pl.ds(row0, 8)], dst_st)
            if weighted:
                pltpu.sync_copy(w_ref.at[pl.ds(base, 1024)], w_st)

            def add_off(r):
                @pl.loop(0, 8)
                def _(g):
                    v = src_st[r, pl.ds(g * 16, 16)]
                    src_st[r, pl.ds(g * 16, 16)] = v + off

            rows = (rows0, rows1)
            sems = (sem0, sem1)
            add_off(0)
            descs = [None, None]
            descs[0] = pltpu.async_copy(tbl_ref.at[src_st.at[0]], rows[0],
                                        sems[0])
            for r in range(1, 8):
                add_off(r)
            for k in range(8):
                buf = k % 2
                nbuf = (k + 1) % 2
                descs[buf].wait()
                if k < 7:
                    descs[nbuf] = pltpu.async_copy(
                        tbl_ref.at[src_st.at[k + 1]], rows[nbuf], sems[nbuf])
                rcur = rows[buf]
                if weighted:
                    @pl.loop(0, 8)
                    def _(g):
                        wv = w_st[pl.ds(k * 128 + g * 16, 16)]
                        for j in range(16):
                            e = g * 16 + j
                            wsc = wv[j]
                            rcur[e, pl.ds(0, 16)] = rcur[e, pl.ds(0, 16)] * wsc
                            rcur[e, pl.ds(16, 16)] = (
                                rcur[e, pl.ds(16, 16)] * wsc)
                pltpu.sync_copy(rcur, acc.at[dst_st.at[k]], add=True)

        plsc.subcore_barrier()

        @pl.loop(0, R // 128)
        def _(k):
            r0 = s * R + k * 128
            pltpu.sync_copy(acc.at[pl.ds(r0, 128)],
                            out_ref.at[pl.ds(c * VDP + r0, 128)])

    return pl.kernel(
        body,
        out_type=jax.ShapeDtypeStruct((2 * VDP, HH), jnp.float32),
        mesh=_MESH,
        compiler_params=pltpu.CompilerParams(use_tc_tiling_on_sc=False),
        scratch_types=[
            pltpu.VMEM_SHARED((VDP, HH), jnp.float32),
            pltpu.VMEM((8, 128), jnp.int32),
            pltpu.VMEM((8, 128), jnp.int32),
            pltpu.VMEM((1024,), jnp.float32),
            pltpu.VMEM((128, HH), jnp.float32),
            pltpu.VMEM((128, HH), jnp.float32),
            pltpu.VMEM((128, HH), jnp.float32),
            pltpu.SemaphoreType.DMA,
            pltpu.SemaphoreType.DMA,
        ],
    )


def _pad1(x, n, val):
    return jnp.pad(x, (0, n - x.shape[0]), constant_values=val)


def _stack_halves(x, vp):
    xp = jnp.pad(x, ((0, vp - x.shape[0]), (0, 0)))
    return jnp.concatenate([xp[:, :HH], xp[:, HH:]], axis=0)


def _unstack(xs, vr, vp):
    return jnp.concatenate([xs[:vr], xs[vp:vp + vr]], axis=1)


_spmv_uu = _make_spmv(False, E_UE_P, NU, NU)
_spmv_ss = _make_spmv(True, E_SP_P, MS, MS)
_spmv_su = _make_spmv(False, E_US_P, MS, NU)
_spmv_us = _make_spmv(False, E_US_P, NU, MS)


def kernel(spot_emb, user_emb, user_spot, spot_edge_index, spot_edge_weight,
           user_edge_index):
    u = _pad1(user_spot[0], E_US_P, N_USER)
    sp = _pad1(user_spot[1], E_US_P, M_SPOT)
    ue_s = _pad1(user_edge_index[0], E_UE_P, N_USER)
    ue_d = _pad1(user_edge_index[1], E_UE_P, N_USER)
    se_s = _pad1(spot_edge_index[0], E_SP_P, M_SPOT)
    se_d = _pad1(spot_edge_index[1], E_SP_P, M_SPOT)
    w = _pad1(spot_edge_weight, E_SP_P, 0.0)

    cu2, cs2, du2, ds2 = _hist_call(u, sp, ue_d, se_d, w)
    cnt_u = (cu2[0] + cu2[1]).reshape(NU)
    cnt_s = (cs2[0] + cs2[1]).reshape(MS)
    deg_u = (du2[0] + du2[1]).reshape(NU)
    deg_s = (ds2[0] + ds2[1]).reshape(MS)
    uinv = jnp.where(cnt_u > 0, lax.rsqrt(jnp.maximum(cnt_u, 1e-30)), 0.0)
    sinv = jnp.where(cnt_s > 0, lax.rsqrt(jnp.maximum(cnt_s, 1e-30)), 0.0)
    dui = jnp.where(deg_u > 0, lax.rsqrt(jnp.maximum(deg_u, 1e-30)), 0.0)
    dsi = jnp.where(deg_s > 0, lax.rsqrt(jnp.maximum(deg_s, 1e-30)), 0.0)
    uinv2 = jnp.concatenate([uinv, uinv])[:, None]
    sinv2 = jnp.concatenate([sinv, sinv])[:, None]
    dui2 = jnp.concatenate([dui, dui])[:, None]
    dsi2 = jnp.concatenate([dsi, dsi])[:, None]

    u2d = u.reshape(E_US_P // 128, 128)
    s2d = sp.reshape(E_US_P // 128, 128)
    ued2d = ue_d.reshape(E_UE_P // 128, 128)
    ues2d = ue_s.reshape(E_UE_P // 128, 128)
    sed2d = se_d.reshape(E_SP_P // 128, 128)
    ses2d = se_s.reshape(E_SP_P // 128, 128)

    user_x = _stack_halves(user_emb, NU)
    spot_x = _stack_halves(spot_emb, MS)

    cat = _spmv_uu(ues2d, ued2d, user_x * dui2)
    user_x = user_x + cat * dui2
    cat = _spmv_ss(ses2d, sed2d, spot_x * dsi2, w)
    spot_x = spot_x + cat * dsi2

    u_out = user_x
    s_out = spot_x
    for _ in range(3):
        nu_ = uinv2 * _spmv_su(s2d, u2d, spot_x * sinv2)
        ns_ = sinv2 * _spmv_us(u2d, s2d, user_x * uinv2)
        user_x = nu_
        spot_x = ns_
        u_out = u_out + user_x
        s_out = s_out + spot_x

    s_out = s_out * 0.25
    u_out = u_out * 0.25
    return _unstack(s_out, M_SPOT, MS), _unstack(u_out, N_USER, NU)


# speedup vs baseline: 12.2642x; 1.2379x over previous
import functools

import jax
import jax.numpy as jnp
from jax import lax
from jax.experimental import pallas as pl
from jax.experimental.pallas import tpu as pltpu
from jax.experimental.pallas import tpu_sc as plsc

N_USER = 27094
M_SPOT = 42852
HID = 64
HH = HID // 2

NU = 28672
MS = 43008

E_US = 1000000
E_SPOT = 685632
E_USER = 541880
E_US_P = 16384 * 62
E_SP_P = 16384 * 42
E_UE_P = 16384 * 34

_MESH = plsc.VectorSubcoreMesh(core_axis_name="c", subcore_axis_name="s")


RU = NU // 16
RS = MS // 16


def _hist_body(u_ref, s_ref, ud_ref, sd_ref, w_ref,
               cu_ref, cs_ref, du_ref, ds_ref,
               h1, h2, st1, st2, wst, ridx, stage, a_cu, a_cs, a_du, a_ds):
    c = lax.axis_index("c")
    s = lax.axis_index("s")
    z16 = jnp.zeros((16,), jnp.float32)
    o16 = jnp.ones((16,), jnp.float32)
    i16 = jnp.arange(16, dtype=jnp.int32)

    @pl.loop(0, NU // 16)
    def _(i):
        h1[pl.ds(i * 16, 16)] = z16

    @pl.loop(0, MS // 16)
    def _(i):
        h2[pl.ds(i * 16, 16)] = z16

    @pl.loop(0, 128)
    def _(r):
        stage[r, pl.ds(0, 16)] = z16

    @pl.loop(0, RS // 128)
    def _(r):
        @pl.loop(0, 8)
        def _(g):
            ridx[r, pl.ds(g * 16, 16)] = i16 + (r * 128 + g * 16)

    @pl.when(s == 0)
    def _():
        for acc, nrow in ((a_cu, RU), (a_cs, RS), (a_du, RU), (a_ds, RS)):
            @pl.loop(0, nrow // 128)
            def _(k):
                pltpu.sync_copy(stage, acc.at[pl.ds(k * 128, 128)])

    plsc.subcore_barrier()

    t = c * 16 + s

    def count_phase(idx_ref, hist, ep, wref=None):
        et = ep // 32

        @pl.loop(0, et // 1024)
        def _(b):
            base = t * et + b * 1024
            pltpu.sync_copy(idx_ref.at[pl.ds(base, 1024)], st1)
            if wref is not None:
                pltpu.sync_copy(wref.at[pl.ds(base, 1024)], wst)

            @pl.loop(0, 64)
            def _(j):
                iv = st1[pl.ds(j * 16, 16)]
                vals = o16 if wref is None else wst[pl.ds(j * 16, 16)]
                plsc.addupdate_scatter(hist, [iv], vals)

    def reduce_phase(hist, acc, nrow):
        @pl.loop(0, nrow // 128)
        def _(k):
            @pl.loop(0, 128)
            def _(r):
                stage[r, pl.ds(0, 16)] = hist[pl.ds((k * 128 + r) * 16, 16)]
            pltpu.sync_copy(stage, acc.at[ridx.at[k]], add=True)

    def zero_hist(hist, n):
        @pl.loop(0, n // 16)
        def _(i):
            hist[pl.ds(i * 16, 16)] = z16

    eta = E_US_P // 32

    @pl.loop(0, eta // 1024)
    def _(b):
        base = t * eta + b * 1024
        pltpu.sync_copy(u_ref.at[pl.ds(base, 1024)], st1)
        pltpu.sync_copy(s_ref.at[pl.ds(base, 1024)], st2)

        @pl.loop(0, 64)
        def _(j):
            plsc.addupdate_scatter(h1, [st1[pl.ds(j * 16, 16)]], o16)
            plsc.addupdate_scatter(h2, [st2[pl.ds(j * 16, 16)]], o16)

    reduce_phase(h1, a_cu, RU)
    reduce_phase(h2, a_cs, RS)

    zero_hist(h1, NU)
    count_phase(ud_ref, h1, E_UE_P)
    reduce_phase(h1, a_du, RU)

    zero_hist(h2, MS)
    count_phase(sd_ref, h2, E_SP_P, w_ref)
    reduce_phase(h2, a_ds, RS)

    plsc.subcore_barrier()

    ru = RU // 16
    rs = RS // 16
    pltpu.sync_copy(a_cu.at[pl.ds(s * ru, ru)], cu_ref.at[c, pl.ds(s * ru, ru)])
    pltpu.sync_copy(a_cs.at[pl.ds(s * rs, rs)], cs_ref.at[c, pl.ds(s * rs, rs)])
    pltpu.sync_copy(a_du.at[pl.ds(s * ru, ru)], du_ref.at[c, pl.ds(s * ru, ru)])
    pltpu.sync_copy(a_ds.at[pl.ds(s * rs, rs)], ds_ref.at[c, pl.ds(s * rs, rs)])


_hist_call = pl.kernel(
    _hist_body,
    out_type=(
        jax.ShapeDtypeStruct((2, RU, 16), jnp.float32),
        jax.ShapeDtypeStruct((2, RS, 16), jnp.float32),
        jax.ShapeDtypeStruct((2, RU, 16), jnp.float32),
        jax.ShapeDtypeStruct((2, RS, 16), jnp.float32),
    ),
    mesh=_MESH,
    compiler_params=pltpu.CompilerParams(needs_layout_passes=False, use_tc_tiling_on_sc=False),
    scratch_types=[
        pltpu.VMEM((NU,), jnp.float32),
        pltpu.VMEM((MS,), jnp.float32),
        pltpu.VMEM((1024,), jnp.int32),
        pltpu.VMEM((1024,), jnp.int32),
        pltpu.VMEM((1024,), jnp.float32),
        pltpu.VMEM((RS // 128, 128), jnp.int32),
        pltpu.VMEM((128, 16), jnp.float32),
        pltpu.VMEM_SHARED((RU, 16), jnp.float32),
        pltpu.VMEM_SHARED((RS, 16), jnp.float32),
        pltpu.VMEM_SHARED((RU, 16), jnp.float32),
        pltpu.VMEM_SHARED((RS, 16), jnp.float32),
    ],
)


def _make_spmv(weighted, EP, VSP, VDP):
    def body(*refs):
        if weighted:
            (src_ref, dst_ref, tbl_ref, w_ref, out_ref,
             acc, src_st, dst_st, w_st, rows0, rows1, zer,
             sem0, sem1) = refs
        else:
            (src_ref, dst_ref, tbl_ref, out_ref,
             acc, src_st, dst_st, w_st, rows0, rows1, zer,
             sem0, sem1) = refs
        c = lax.axis_index("c")
        s = lax.axis_index("s")
        z16 = jnp.zeros((16,), jnp.float32)

        @pl.loop(0, 128)
        def _(r):
            zer[r, pl.ds(0, 16)] = z16
            zer[r, pl.ds(16, 16)] = z16

        R = VDP // 16

        @pl.loop(0, R // 128)
        def _(k):
            pltpu.sync_copy(zer, acc.at[pl.ds(s * R + k * 128, 128)])

        plsc.subcore_barrier()

        ET = EP // 16
        ETR = ET // 128
        off = c * VSP

        @pl.loop(0, ET // 1024)
        def _(b):
            base = s * ET + b * 1024
            row0 = s * (ET // 512) + b * 2
            pltpu.sync_copy(src_ref.at[pl.ds(base, 1024)], src_st)
            pltpu.sync_copy(dst_ref.at[pl.ds(row0, 2)], dst_st)
            if weighted:
                pltpu.sync_copy(w_ref.at[pl.ds(base, 1024)], w_st)

            def add_off(half):
                @pl.loop(0, 32)
                def _(g):
                    i0 = half * 512 + g * 16
                    v = src_st[pl.ds(i0, 16)]
                    src_st[pl.ds(i0, 16)] = v + off

            rows = (rows0, rows1)
            sems = (sem0, sem1)
            add_off(0)
            descs = [None, None]
            descs[0] = pltpu.async_copy(tbl_ref.at[src_st.at[pl.ds(0, 512)]],
                                        rows[0], sems[0])
            add_off(1)
            for k in range(2):
                buf = k % 2
                nbuf = (k + 1) % 2
                descs[buf].wait()
                if k < 1:
                    descs[nbuf] = pltpu.async_copy(
                        tbl_ref.at[src_st.at[pl.ds(512, 512)]], rows[nbuf],
                        sems[nbuf])
                rcur = rows[buf]
                if weighted:
                    @pl.loop(0, 32)
                    def _(g):
                        wv = w_st[pl.ds(k * 512 + g * 16, 16)]
                        for j in range(16):
                            e = g * 16 + j
                            wsc = wv[j]
                            rcur[e, pl.ds(0, 16)] = rcur[e, pl.ds(0, 16)] * wsc
                            rcur[e, pl.ds(16, 16)] = (
                                rcur[e, pl.ds(16, 16)] * wsc)
                pltpu.sync_copy(rcur, acc.at[dst_st.at[k]], add=True)

        plsc.subcore_barrier()

        @pl.loop(0, R // 128)
        def _(k):
            r0 = s * R + k * 128
            pltpu.sync_copy(acc.at[pl.ds(r0, 128)],
                            out_ref.at[pl.ds(c * VDP + r0, 128)])

    return pl.kernel(
        body,
        out_type=jax.ShapeDtypeStruct((2 * VDP, HH), jnp.float32),
        mesh=_MESH,
        compiler_params=pltpu.CompilerParams(use_tc_tiling_on_sc=False),
        scratch_types=[
            pltpu.VMEM_SHARED((VDP, HH), jnp.float32),
            pltpu.VMEM((1024,), jnp.int32),
            pltpu.VMEM((2, 512), jnp.int32),
            pltpu.VMEM((1024,), jnp.float32),
            pltpu.VMEM((512, HH), jnp.float32),
            pltpu.VMEM((512, HH), jnp.float32),
            pltpu.VMEM((128, HH), jnp.float32),
            pltpu.SemaphoreType.DMA,
            pltpu.SemaphoreType.DMA,
        ],
    )


def _pad1(x, n, val):
    return jnp.pad(x, (0, n - x.shape[0]), constant_values=val)


def _stack_halves(x, vp):
    xp = jnp.pad(x, ((0, vp - x.shape[0]), (0, 0)))
    return jnp.concatenate([xp[:, :HH], xp[:, HH:]], axis=0)


def _unstack(xs, vr, vp):
    return jnp.concatenate([xs[:vr], xs[vp:vp + vr]], axis=1)


_spmv_uu = _make_spmv(False, E_UE_P, NU, NU)
_spmv_ss = _make_spmv(True, E_SP_P, MS, MS)
_spmv_su = _make_spmv(False, E_US_P, MS, NU)
_spmv_us = _make_spmv(False, E_US_P, NU, MS)


def kernel(spot_emb, user_emb, user_spot, spot_edge_index, spot_edge_weight,
           user_edge_index):
    u = _pad1(user_spot[0], E_US_P, N_USER)
    sp = _pad1(user_spot[1], E_US_P, M_SPOT)
    ue_s = _pad1(user_edge_index[0], E_UE_P, N_USER)
    ue_d = _pad1(user_edge_index[1], E_UE_P, N_USER)
    se_s = _pad1(spot_edge_index[0], E_SP_P, M_SPOT)
    se_d = _pad1(spot_edge_index[1], E_SP_P, M_SPOT)
    w = _pad1(spot_edge_weight, E_SP_P, 0.0)

    cu2, cs2, du2, ds2 = _hist_call(u, sp, ue_d, se_d, w)
    cnt_u = (cu2[0] + cu2[1]).reshape(NU)
    cnt_s = (cs2[0] + cs2[1]).reshape(MS)
    deg_u = (du2[0] + du2[1]).reshape(NU)
    deg_s = (ds2[0] + ds2[1]).reshape(MS)
    uinv = jnp.where(cnt_u > 0, lax.rsqrt(jnp.maximum(cnt_u, 1e-30)), 0.0)
    sinv = jnp.where(cnt_s > 0, lax.rsqrt(jnp.maximum(cnt_s, 1e-30)), 0.0)
    dui = jnp.where(deg_u > 0, lax.rsqrt(jnp.maximum(deg_u, 1e-30)), 0.0)
    dsi = jnp.where(deg_s > 0, lax.rsqrt(jnp.maximum(deg_s, 1e-30)), 0.0)
    uinv2 = jnp.concatenate([uinv, uinv])[:, None]
    sinv2 = jnp.concatenate([sinv, sinv])[:, None]
    dui2 = jnp.concatenate([dui, dui])[:, None]
    dsi2 = jnp.concatenate([dsi, dsi])[:, None]

    u2d = u.reshape(E_US_P // 512, 512)
    s2d = sp.reshape(E_US_P // 512, 512)
    ued2d = ue_d.reshape(E_UE_P // 512, 512)
    sed2d = se_d.reshape(E_SP_P // 512, 512)

    user_x = _stack_halves(user_emb, NU)
    spot_x = _stack_halves(spot_emb, MS)

    cat = _spmv_uu(ue_s, ued2d, user_x * dui2)
    user_x = user_x + cat * dui2
    cat = _spmv_ss(se_s, sed2d, spot_x * dsi2, w)
    spot_x = spot_x + cat * dsi2

    u_out = user_x
    s_out = spot_x
    for _ in range(3):
        nu_ = uinv2 * _spmv_su(sp, u2d, spot_x * sinv2)
        ns_ = sinv2 * _spmv_us(u, s2d, user_x * uinv2)
        user_x = nu_
        spot_x = ns_
        u_out = u_out + user_x
        s_out = s_out + spot_x

    s_out = s_out * 0.25
    u_out = u_out * 0.25
    return _unstack(s_out, M_SPOT, MS), _unstack(u_out, N_USER, NU)


# speedup vs baseline: 13.9499x; 1.1374x over previous
import functools

import jax
import jax.numpy as jnp
from jax import lax
from jax.experimental import pallas as pl
from jax.experimental.pallas import tpu as pltpu
from jax.experimental.pallas import tpu_sc as plsc

N_USER = 27094
M_SPOT = 42852
HID = 64
HH = HID // 2

NU = 28672
MS = 43008

E_US = 1000000
E_SPOT = 685632
E_USER = 541880
E_US_P = 16384 * 62
E_SP_P = 16384 * 42
E_UE_P = 16384 * 34

_MESH = plsc.VectorSubcoreMesh(core_axis_name="c", subcore_axis_name="s")


RU = NU // 16
RS = MS // 16


def _hist_body(u_ref, s_ref, ud_ref, sd_ref, w_ref,
               cu_ref, cs_ref, du_ref, ds_ref,
               h1, h2, st1, st2, wst, ridx, stage, a_cu, a_cs, a_du, a_ds):
    c = lax.axis_index("c")
    s = lax.axis_index("s")
    z16 = jnp.zeros((16,), jnp.float32)
    o16 = jnp.ones((16,), jnp.float32)
    i16 = jnp.arange(16, dtype=jnp.int32)

    @pl.loop(0, NU // 16)
    def _(i):
        h1[pl.ds(i * 16, 16)] = z16

    @pl.loop(0, MS // 16)
    def _(i):
        h2[pl.ds(i * 16, 16)] = z16

    @pl.loop(0, 128)
    def _(r):
        stage[r, pl.ds(0, 16)] = z16

    @pl.loop(0, RS // 128)
    def _(r):
        @pl.loop(0, 8)
        def _(g):
            ridx[r, pl.ds(g * 16, 16)] = i16 + (r * 128 + g * 16)

    @pl.when(s == 0)
    def _():
        for acc, nrow in ((a_cu, RU), (a_cs, RS), (a_du, RU), (a_ds, RS)):
            @pl.loop(0, nrow // 128)
            def _(k):
                pltpu.sync_copy(stage, acc.at[pl.ds(k * 128, 128)])

    plsc.subcore_barrier()

    t = c * 16 + s

    def count_phase(idx_ref, hist, ep, wref=None):
        et = ep // 32

        @pl.loop(0, et // 1024)
        def _(b):
            base = t * et + b * 1024
            pltpu.sync_copy(idx_ref.at[pl.ds(base, 1024)], st1)
            if wref is not None:
                pltpu.sync_copy(wref.at[pl.ds(base, 1024)], wst)

            @pl.loop(0, 64)
            def _(j):
                iv = st1[pl.ds(j * 16, 16)]
                vals = o16 if wref is None else wst[pl.ds(j * 16, 16)]
                plsc.addupdate_scatter(hist, [iv], vals)

    def reduce_phase(hist, acc, nrow):
        @pl.loop(0, nrow // 128)
        def _(k):
            @pl.loop(0, 128)
            def _(r):
                stage[r, pl.ds(0, 16)] = hist[pl.ds((k * 128 + r) * 16, 16)]
            pltpu.sync_copy(stage, acc.at[ridx.at[k]], add=True)

    def zero_hist(hist, n):
        @pl.loop(0, n // 16)
        def _(i):
            hist[pl.ds(i * 16, 16)] = z16

    eta = E_US_P // 32

    @pl.loop(0, eta // 1024)
    def _(b):
        base = t * eta + b * 1024
        pltpu.sync_copy(u_ref.at[pl.ds(base, 1024)], st1)
        pltpu.sync_copy(s_ref.at[pl.ds(base, 1024)], st2)

        @pl.loop(0, 64)
        def _(j):
            plsc.addupdate_scatter(h1, [st1[pl.ds(j * 16, 16)]], o16)
            plsc.addupdate_scatter(h2, [st2[pl.ds(j * 16, 16)]], o16)

    reduce_phase(h1, a_cu, RU)
    reduce_phase(h2, a_cs, RS)

    zero_hist(h1, NU)
    count_phase(ud_ref, h1, E_UE_P)
    reduce_phase(h1, a_du, RU)

    zero_hist(h2, MS)
    count_phase(sd_ref, h2, E_SP_P, w_ref)
    reduce_phase(h2, a_ds, RS)

    plsc.subcore_barrier()

    ru = RU // 16
    rs = RS // 16
    pltpu.sync_copy(a_cu.at[pl.ds(s * ru, ru)], cu_ref.at[c, pl.ds(s * ru, ru)])
    pltpu.sync_copy(a_cs.at[pl.ds(s * rs, rs)], cs_ref.at[c, pl.ds(s * rs, rs)])
    pltpu.sync_copy(a_du.at[pl.ds(s * ru, ru)], du_ref.at[c, pl.ds(s * ru, ru)])
    pltpu.sync_copy(a_ds.at[pl.ds(s * rs, rs)], ds_ref.at[c, pl.ds(s * rs, rs)])


_hist_call = pl.kernel(
    _hist_body,
    out_type=(
        jax.ShapeDtypeStruct((2, RU, 16), jnp.float32),
        jax.ShapeDtypeStruct((2, RS, 16), jnp.float32),
        jax.ShapeDtypeStruct((2, RU, 16), jnp.float32),
        jax.ShapeDtypeStruct((2, RS, 16), jnp.float32),
    ),
    mesh=_MESH,
    compiler_params=pltpu.CompilerParams(needs_layout_passes=False, use_tc_tiling_on_sc=False),
    scratch_types=[
        pltpu.VMEM((NU,), jnp.float32),
        pltpu.VMEM((MS,), jnp.float32),
        pltpu.VMEM((1024,), jnp.int32),
        pltpu.VMEM((1024,), jnp.int32),
        pltpu.VMEM((1024,), jnp.float32),
        pltpu.VMEM((RS // 128, 128), jnp.int32),
        pltpu.VMEM((128, 16), jnp.float32),
        pltpu.VMEM_SHARED((RU, 16), jnp.float32),
        pltpu.VMEM_SHARED((RS, 16), jnp.float32),
        pltpu.VMEM_SHARED((RU, 16), jnp.float32),
        pltpu.VMEM_SHARED((RS, 16), jnp.float32),
    ],
)


def _make_spmv(weighted, EP, VSP, VDP):
    def body(*refs):
        if weighted:
            (src_ref, dst_ref, tbl_ref, w_ref, out_ref,
             acc, src_st, dst_st, w_st, r0, r1, r2, r3, zer,
             g0, g1, g2, g3, s0, s1, s2, s3) = refs
        else:
            (src_ref, dst_ref, tbl_ref, out_ref,
             acc, src_st, dst_st, w_st, r0, r1, r2, r3, zer,
             g0, g1, g2, g3, s0, s1, s2, s3) = refs
        c = lax.axis_index("c")
        s = lax.axis_index("s")
        z16 = jnp.zeros((16,), jnp.float32)
        rowsb = (r0, r1, r2, r3)
        gsem = (g0, g1, g2, g3)
        ssem = (s0, s1, s2, s3)

        @pl.loop(0, 64)
        def _(r):
            zer[r, pl.ds(0, 16)] = z16
            zer[r, pl.ds(16, 16)] = z16

        R = VDP // 16

        @pl.loop(0, R // 64)
        def _(k):
            pltpu.sync_copy(zer, acc.at[pl.ds(s * R + k * 64, 64)])

        plsc.subcore_barrier()

        ET = EP // 16
        NB = ET // 1024

        def stage(b, half):
            base = s * ET + b * 1024
            pltpu.sync_copy(src_ref.at[c, pl.ds(base, 1024)], src_st)
            pltpu.sync_copy(dst_ref.at[pl.ds(s * (ET // 256) + b * 4, 4)],
                            dst_st.at[pl.ds(half * 4, 4)])
            if weighted:
                pltpu.sync_copy(w_ref.at[pl.ds(base, 1024)], w_st)

        def gissue(k):
            return pltpu.async_copy(
                tbl_ref.at[src_st.at[pl.ds(k * 256, 256)]], rowsb[k], gsem[k])

        stage(0, 0)
        gd = [gissue(k) for k in range(4)]

        @pl.loop(0, NB)
        def _(b):
            half = b & 1
            sd = []
            for k in range(4):
                gd[k].wait()
                rcur = rowsb[k]
                if weighted:
                    @pl.loop(0, 16)
                    def _(g):
                        wv = w_st[pl.ds(k * 256 + g * 16, 16)]
                        for j in range(16):
                            e = g * 16 + j
                            wsc = wv[j]
                            rcur[e, pl.ds(0, 16)] = rcur[e, pl.ds(0, 16)] * wsc
                            rcur[e, pl.ds(16, 16)] = (
                                rcur[e, pl.ds(16, 16)] * wsc)
                sd.append(pltpu.async_copy(
                    rcur, acc.at[dst_st.at[half * 4 + k]], ssem[k]))

            @pl.when(b < NB - 1)
            def _():
                stage(b + 1, 1 - half)
                for k in range(4):
                    sd[k].wait()
                    gissue(k)

        for k in range(4):
            pltpu.make_async_copy(rowsb[k], acc.at[dst_st.at[k]],
                                  ssem[k]).wait()

        plsc.subcore_barrier()

        @pl.loop(0, R // 128)
        def _(k):
            r0_ = s * R + k * 128
            pltpu.sync_copy(acc.at[pl.ds(r0_, 128)],
                            out_ref.at[pl.ds(c * VDP + r0_, 128)])

    return pl.kernel(
        body,
        out_type=jax.ShapeDtypeStruct((2 * VDP, HH), jnp.float32),
        mesh=_MESH,
        compiler_params=pltpu.CompilerParams(use_tc_tiling_on_sc=False),
        scratch_types=[
            pltpu.VMEM_SHARED((VDP, HH), jnp.float32),
            pltpu.VMEM((1024,), jnp.int32),
            pltpu.VMEM((8, 256), jnp.int32),
            pltpu.VMEM((1024,), jnp.float32),
            pltpu.VMEM((256, HH), jnp.float32),
            pltpu.VMEM((256, HH), jnp.float32),
            pltpu.VMEM((256, HH), jnp.float32),
            pltpu.VMEM((256, HH), jnp.float32),
            pltpu.VMEM((64, HH), jnp.float32),
            pltpu.SemaphoreType.DMA,
            pltpu.SemaphoreType.DMA,
            pltpu.SemaphoreType.DMA,
            pltpu.SemaphoreType.DMA,
            pltpu.SemaphoreType.DMA,
            pltpu.SemaphoreType.DMA,
            pltpu.SemaphoreType.DMA,
            pltpu.SemaphoreType.DMA,
        ],
    )


def _pad1(x, n, val):
    return jnp.pad(x, (0, n - x.shape[0]), constant_values=val)


def _stack_halves(x, vp):
    xp = jnp.pad(x, ((0, vp - x.shape[0]), (0, 0)))
    return jnp.concatenate([xp[:, :HH], xp[:, HH:]], axis=0)


def _unstack(xs, vr, vp):
    return jnp.concatenate([xs[:vr], xs[vp:vp + vr]], axis=1)


_spmv_uu = _make_spmv(False, E_UE_P, NU, NU)
_spmv_ss = _make_spmv(True, E_SP_P, MS, MS)
_spmv_su = _make_spmv(False, E_US_P, MS, NU)
_spmv_us = _make_spmv(False, E_US_P, NU, MS)


def kernel(spot_emb, user_emb, user_spot, spot_edge_index, spot_edge_weight,
           user_edge_index):
    u = _pad1(user_spot[0], E_US_P, N_USER)
    sp = _pad1(user_spot[1], E_US_P, M_SPOT)
    ue_s = _pad1(user_edge_index[0], E_UE_P, N_USER)
    ue_d = _pad1(user_edge_index[1], E_UE_P, N_USER)
    se_s = _pad1(spot_edge_index[0], E_SP_P, M_SPOT)
    se_d = _pad1(spot_edge_index[1], E_SP_P, M_SPOT)
    w = _pad1(spot_edge_weight, E_SP_P, 0.0)

    cu2, cs2, du2, ds2 = _hist_call(u, sp, ue_d, se_d, w)
    cnt_u = (cu2[0] + cu2[1]).reshape(NU)
    cnt_s = (cs2[0] + cs2[1]).reshape(MS)
    deg_u = (du2[0] + du2[1]).reshape(NU)
    deg_s = (ds2[0] + ds2[1]).reshape(MS)
    uinv = jnp.where(cnt_u > 0, lax.rsqrt(jnp.maximum(cnt_u, 1e-30)), 0.0)
    sinv = jnp.where(cnt_s > 0, lax.rsqrt(jnp.maximum(cnt_s, 1e-30)), 0.0)
    dui = jnp.where(deg_u > 0, lax.rsqrt(jnp.maximum(deg_u, 1e-30)), 0.0)
    dsi = jnp.where(deg_s > 0, lax.rsqrt(jnp.maximum(deg_s, 1e-30)), 0.0)
    uinv2 = jnp.concatenate([uinv, uinv])[:, None]
    sinv2 = jnp.concatenate([sinv, sinv])[:, None]
    dui2 = jnp.concatenate([dui, dui])[:, None]
    dsi2 = jnp.concatenate([dsi, dsi])[:, None]

    u2d = u.reshape(E_US_P // 256, 256)
    s2d = sp.reshape(E_US_P // 256, 256)
    ued2d = ue_d.reshape(E_UE_P // 256, 256)
    sed2d = se_d.reshape(E_SP_P // 256, 256)

    user_x = _stack_halves(user_emb, NU)
    spot_x = _stack_halves(spot_emb, MS)

    ue_s2 = jnp.stack([ue_s, ue_s + NU])
    se_s2 = jnp.stack([se_s, se_s + MS])
    u_2 = jnp.stack([u, u + NU])
    sp_2 = jnp.stack([sp, sp + MS])

    cat = _spmv_uu(ue_s2, ued2d, user_x * dui2)
    user_x = user_x + cat * dui2
    cat = _spmv_ss(se_s2, sed2d, spot_x * dsi2, w)
    spot_x = spot_x + cat * dsi2

    u_out = user_x
    s_out = spot_x
    for _ in range(3):
        nu_ = uinv2 * _spmv_su(sp_2, u2d, spot_x * sinv2)
        ns_ = sinv2 * _spmv_us(u_2, s2d, user_x * uinv2)
        user_x = nu_
        spot_x = ns_
        u_out = u_out + user_x
        s_out = s_out + spot_x

    s_out = s_out * 0.25
    u_out = u_out * 0.25
    return _unstack(s_out, M_SPOT, MS), _unstack(u_out, N_USER, NU)


# speedup vs baseline: 13.9789x; 1.0021x over previous
import functools

import jax
import jax.numpy as jnp
from jax import lax
from jax.experimental import pallas as pl
from jax.experimental.pallas import tpu as pltpu
from jax.experimental.pallas import tpu_sc as plsc

N_USER = 27094
M_SPOT = 42852
HID = 64
HH = HID // 2

NU = 28672
MS = 43008

E_US = 1000000
E_SPOT = 685632
E_USER = 541880
E_US_P = 16384 * 62
E_SP_P = 16384 * 42
E_UE_P = 16384 * 34

_MESH = plsc.VectorSubcoreMesh(core_axis_name="c", subcore_axis_name="s")


RU = NU // 16
RS = MS // 16


def _hist_body(u_ref, s_ref, ud_ref, sd_ref, w_ref,
               cu_ref, cs_ref, du_ref, ds_ref,
               h1, h2, st1, st2, wst, ridx, stage, a_cu, a_cs, a_du, a_ds):
    c = lax.axis_index("c")
    s = lax.axis_index("s")
    z16 = jnp.zeros((16,), jnp.float32)
    o16 = jnp.ones((16,), jnp.float32)
    i16 = jnp.arange(16, dtype=jnp.int32)

    @pl.loop(0, NU // 16)
    def _(i):
        h1[pl.ds(i * 16, 16)] = z16

    @pl.loop(0, MS // 16)
    def _(i):
        h2[pl.ds(i * 16, 16)] = z16

    @pl.loop(0, 128)
    def _(r):
        stage[r, pl.ds(0, 16)] = z16

    @pl.loop(0, RS // 128)
    def _(r):
        @pl.loop(0, 8)
        def _(g):
            ridx[r, pl.ds(g * 16, 16)] = i16 + (r * 128 + g * 16)

    @pl.when(s == 0)
    def _():
        for acc, nrow in ((a_cu, RU), (a_cs, RS), (a_du, RU), (a_ds, RS)):
            @pl.loop(0, nrow // 128)
            def _(k):
                pltpu.sync_copy(stage, acc.at[pl.ds(k * 128, 128)])

    plsc.subcore_barrier()

    t = c * 16 + s

    def count_phase(idx_ref, hist, ep, wref=None):
        et = ep // 32

        @pl.loop(0, et // 1024)
        def _(b):
            base = t * et + b * 1024
            pltpu.sync_copy(idx_ref.at[pl.ds(base, 1024)], st1)
            if wref is not None:
                pltpu.sync_copy(wref.at[pl.ds(base, 1024)], wst)

            @pl.loop(0, 64)
            def _(j):
                iv = st1[pl.ds(j * 16, 16)]
                vals = o16 if wref is None else wst[pl.ds(j * 16, 16)]
                plsc.addupdate_scatter(hist, [iv], vals)

    def reduce_phase(hist, acc, nrow):
        @pl.loop(0, nrow // 128)
        def _(k):
            @pl.loop(0, 128)
            def _(r):
                stage[r, pl.ds(0, 16)] = hist[pl.ds((k * 128 + r) * 16, 16)]
            pltpu.sync_copy(stage, acc.at[ridx.at[k]], add=True)

    def zero_hist(hist, n):
        @pl.loop(0, n // 16)
        def _(i):
            hist[pl.ds(i * 16, 16)] = z16

    eta = E_US_P // 32

    @pl.loop(0, eta // 1024)
    def _(b):
        base = t * eta + b * 1024
        pltpu.sync_copy(u_ref.at[pl.ds(base, 1024)], st1)
        pltpu.sync_copy(s_ref.at[pl.ds(base, 1024)], st2)

        @pl.loop(0, 64)
        def _(j):
            plsc.addupdate_scatter(h1, [st1[pl.ds(j * 16, 16)]], o16)
            plsc.addupdate_scatter(h2, [st2[pl.ds(j * 16, 16)]], o16)

    reduce_phase(h1, a_cu, RU)
    reduce_phase(h2, a_cs, RS)

    zero_hist(h1, NU)
    count_phase(ud_ref, h1, E_UE_P)
    reduce_phase(h1, a_du, RU)

    zero_hist(h2, MS)
    count_phase(sd_ref, h2, E_SP_P, w_ref)
    reduce_phase(h2, a_ds, RS)

    plsc.subcore_barrier()

    ru = RU // 16
    rs = RS // 16
    pltpu.sync_copy(a_cu.at[pl.ds(s * ru, ru)], cu_ref.at[c, pl.ds(s * ru, ru)])
    pltpu.sync_copy(a_cs.at[pl.ds(s * rs, rs)], cs_ref.at[c, pl.ds(s * rs, rs)])
    pltpu.sync_copy(a_du.at[pl.ds(s * ru, ru)], du_ref.at[c, pl.ds(s * ru, ru)])
    pltpu.sync_copy(a_ds.at[pl.ds(s * rs, rs)], ds_ref.at[c, pl.ds(s * rs, rs)])


_hist_call = pl.kernel(
    _hist_body,
    out_type=(
        jax.ShapeDtypeStruct((2, RU, 16), jnp.float32),
        jax.ShapeDtypeStruct((2, RS, 16), jnp.float32),
        jax.ShapeDtypeStruct((2, RU, 16), jnp.float32),
        jax.ShapeDtypeStruct((2, RS, 16), jnp.float32),
    ),
    mesh=_MESH,
    compiler_params=pltpu.CompilerParams(needs_layout_passes=False, use_tc_tiling_on_sc=False),
    scratch_types=[
        pltpu.VMEM((NU,), jnp.float32),
        pltpu.VMEM((MS,), jnp.float32),
        pltpu.VMEM((1024,), jnp.int32),
        pltpu.VMEM((1024,), jnp.int32),
        pltpu.VMEM((1024,), jnp.float32),
        pltpu.VMEM((RS // 128, 128), jnp.int32),
        pltpu.VMEM((128, 16), jnp.float32),
        pltpu.VMEM_SHARED((RU, 16), jnp.float32),
        pltpu.VMEM_SHARED((RS, 16), jnp.float32),
        pltpu.VMEM_SHARED((RU, 16), jnp.float32),
        pltpu.VMEM_SHARED((RS, 16), jnp.float32),
    ],
)


def _make_spmv(weighted, EP, VSP, VDP):
    def body(*refs):
        if weighted:
            (src_ref, dst_ref, tbl_ref, w_ref, out_ref,
             acc, src_st, dst_st, w_st, rA, rB, zer, gsA, gsB) = refs
        else:
            (src_ref, dst_ref, tbl_ref, out_ref,
             acc, src_st, dst_st, w_st, rA, rB, zer, gsA, gsB) = refs
        c = lax.axis_index("c")
        s = lax.axis_index("s")
        z16 = jnp.zeros((16,), jnp.float32)
        rowsb = (rA, rB)
        gsem = (gsA, gsB)

        @pl.loop(0, 64)
        def _(r):
            zer[r, pl.ds(0, 16)] = z16
            zer[r, pl.ds(16, 16)] = z16

        R = VDP // 16

        @pl.loop(0, R // 64)
        def _(k):
            pltpu.sync_copy(zer, acc.at[pl.ds(s * R + k * 64, 64)])

        plsc.subcore_barrier()

        ET = EP // 16
        NB = ET // 1024
        NB2 = NB // 2

        def stage(b, half):
            base = s * ET + b * 1024
            pltpu.sync_copy(src_ref.at[c, pl.ds(base, 1024)], src_st)
            pltpu.sync_copy(dst_ref.at[pl.ds(s * (ET // 512) + b * 2, 2)],
                            dst_st.at[pl.ds(half * 2, 2)])
            if weighted:
                pltpu.sync_copy(w_ref.at[pl.ds(base, 1024)], w_st)

        def gis(k, buf):
            return pltpu.async_copy(
                tbl_ref.at[src_st.at[pl.ds(k * 512, 512)]], rowsb[buf],
                gsem[buf])

        def scale(buf, k):
            if not weighted:
                return
            rcur = rowsb[buf]

            @pl.loop(0, 32)
            def _(g):
                wv = w_st[pl.ds(k * 512 + g * 16, 16)]
                for j in range(16):
                    e = g * 16 + j
                    wsc = wv[j]
                    rcur[e, pl.ds(0, 16)] = rcur[e, pl.ds(0, 16)] * wsc
                    rcur[e, pl.ds(16, 16)] = rcur[e, pl.ds(16, 16)] * wsc

        def scat(half, k, buf):
            pltpu.sync_copy(rowsb[buf], acc.at[dst_st.at[half * 2 + k]],
                            add=True)

        stage(0, 0)
        gd = [gis(0, 0), gis(1, 1)]

        @pl.loop(0, NB2)
        def _(b2):
            b0 = b2 * 2
            gd[0].wait()
            scale(0, 0)
            scat(0, 0, 0)
            gd[1].wait()
            scale(1, 1)
            stage(b0 + 1, 1)
            gis(0, 0)
            scat(0, 1, 1)
            gis(1, 1)
            gd[0].wait()
            scale(0, 0)
            scat(1, 0, 0)

            @pl.when(b2 < NB2 - 1)
            def _():
                gd[1].wait()
                scale(1, 1)
                stage(b0 + 2, 0)
                gis(0, 0)
                scat(1, 1, 1)
                gis(1, 1)

            @pl.when(b2 == NB2 - 1)
            def _():
                gd[1].wait()
                scale(1, 1)
                scat(1, 1, 1)

        plsc.subcore_barrier()

        @pl.loop(0, R // 128)
        def _(k):
            r0_ = s * R + k * 128
            pltpu.sync_copy(acc.at[pl.ds(r0_, 128)],
                            out_ref.at[pl.ds(c * VDP + r0_, 128)])

    return pl.kernel(
        body,
        out_type=jax.ShapeDtypeStruct((2 * VDP, HH), jnp.float32),
        mesh=_MESH,
        compiler_params=pltpu.CompilerParams(use_tc_tiling_on_sc=False),
        scratch_types=[
            pltpu.VMEM_SHARED((VDP, HH), jnp.float32),
            pltpu.VMEM((1024,), jnp.int32),
            pltpu.VMEM((4, 512), jnp.int32),
            pltpu.VMEM((1024,), jnp.float32),
            pltpu.VMEM((512, HH), jnp.float32),
            pltpu.VMEM((512, HH), jnp.float32),
            pltpu.VMEM((64, HH), jnp.float32),
            pltpu.SemaphoreType.DMA,
            pltpu.SemaphoreType.DMA,
        ],
    )


def _pad1(x, n, val):
    return jnp.pad(x, (0, n - x.shape[0]), constant_values=val)


def _stack_halves(x, vp):
    xp = jnp.pad(x, ((0, vp - x.shape[0]), (0, 0)))
    return jnp.concatenate([xp[:, :HH], xp[:, HH:]], axis=0)


def _unstack(xs, vr, vp):
    return jnp.concatenate([xs[:vr], xs[vp:vp + vr]], axis=1)


_spmv_uu = _make_spmv(False, E_UE_P, NU, NU)
_spmv_ss = _make_spmv(True, E_SP_P, MS, MS)
_spmv_su = _make_spmv(False, E_US_P, MS, NU)
_spmv_us = _make_spmv(False, E_US_P, NU, MS)


def kernel(spot_emb, user_emb, user_spot, spot_edge_index, spot_edge_weight,
           user_edge_index):
    u = _pad1(user_spot[0], E_US_P, N_USER)
    sp = _pad1(user_spot[1], E_US_P, M_SPOT)
    ue_s = _pad1(user_edge_index[0], E_UE_P, N_USER)
    ue_d = _pad1(user_edge_index[1], E_UE_P, N_USER)
    se_s = _pad1(spot_edge_index[0], E_SP_P, M_SPOT)
    se_d = _pad1(spot_edge_index[1], E_SP_P, M_SPOT)
    w = _pad1(spot_edge_weight, E_SP_P, 0.0)

    cu2, cs2, du2, ds2 = _hist_call(u, sp, ue_d, se_d, w)
    cnt_u = (cu2[0] + cu2[1]).reshape(NU)
    cnt_s = (cs2[0] + cs2[1]).reshape(MS)
    deg_u = (du2[0] + du2[1]).reshape(NU)
    deg_s = (ds2[0] + ds2[1]).reshape(MS)
    uinv = jnp.where(cnt_u > 0, lax.rsqrt(jnp.maximum(cnt_u, 1e-30)), 0.0)
    sinv = jnp.where(cnt_s > 0, lax.rsqrt(jnp.maximum(cnt_s, 1e-30)), 0.0)
    dui = jnp.where(deg_u > 0, lax.rsqrt(jnp.maximum(deg_u, 1e-30)), 0.0)
    dsi = jnp.where(deg_s > 0, lax.rsqrt(jnp.maximum(deg_s, 1e-30)), 0.0)
    uinv2 = jnp.concatenate([uinv, uinv])[:, None]
    sinv2 = jnp.concatenate([sinv, sinv])[:, None]
    dui2 = jnp.concatenate([dui, dui])[:, None]
    dsi2 = jnp.concatenate([dsi, dsi])[:, None]

    u2d = u.reshape(E_US_P // 512, 512)
    s2d = sp.reshape(E_US_P // 512, 512)
    ued2d = ue_d.reshape(E_UE_P // 512, 512)
    sed2d = se_d.reshape(E_SP_P // 512, 512)

    user_x = _stack_halves(user_emb, NU)
    spot_x = _stack_halves(spot_emb, MS)

    ue_s2 = jnp.stack([ue_s, ue_s + NU])
    se_s2 = jnp.stack([se_s, se_s + MS])
    u_2 = jnp.stack([u, u + NU])
    sp_2 = jnp.stack([sp, sp + MS])

    cat = _spmv_uu(ue_s2, ued2d, user_x * dui2)
    user_x = user_x + cat * dui2
    cat = _spmv_ss(se_s2, sed2d, spot_x * dsi2, w)
    spot_x = spot_x + cat * dsi2

    u_out = user_x
    s_out = spot_x
    for _ in range(3):
        nu_ = uinv2 * _spmv_su(sp_2, u2d, spot_x * sinv2)
        ns_ = sinv2 * _spmv_us(u_2, s2d, user_x * uinv2)
        user_x = nu_
        spot_x = ns_
        u_out = u_out + user_x
        s_out = s_out + spot_x

    s_out = s_out * 0.25
    u_out = u_out * 0.25
    return _unstack(s_out, M_SPOT, MS), _unstack(u_out, N_USER, NU)


# speedup vs baseline: 14.1723x; 1.0138x over previous
import functools

import jax
import jax.numpy as jnp
from jax import lax
from jax.experimental import pallas as pl
from jax.experimental.pallas import tpu as pltpu
from jax.experimental.pallas import tpu_sc as plsc

N_USER = 27094
M_SPOT = 42852
HID = 64
HH = HID // 2

NU = 28672
MS = 43008

E_US = 1000000
E_SPOT = 685632
E_USER = 541880
E_US_P = 16384 * 62
E_SP_P = 16384 * 42
E_UE_P = 16384 * 34

_MESH = plsc.VectorSubcoreMesh(core_axis_name="c", subcore_axis_name="s")


RU = NU // 16
RS = MS // 16


def _hist_body(u_ref, s_ref, ud_ref, sd_ref, w_ref,
               cu_ref, cs_ref, du_ref, ds_ref,
               h1, h2, st1, st2, wst, ridx, stage, a_cu, a_cs, a_du, a_ds):
    c = lax.axis_index("c")
    s = lax.axis_index("s")
    z16 = jnp.zeros((16,), jnp.float32)
    o16 = jnp.ones((16,), jnp.float32)
    i16 = jnp.arange(16, dtype=jnp.int32)

    @pl.loop(0, NU // 16)
    def _(i):
        h1[pl.ds(i * 16, 16)] = z16

    @pl.loop(0, MS // 16)
    def _(i):
        h2[pl.ds(i * 16, 16)] = z16

    @pl.loop(0, 128)
    def _(r):
        stage[r, pl.ds(0, 16)] = z16

    @pl.loop(0, RS // 128)
    def _(r):
        @pl.loop(0, 8)
        def _(g):
            ridx[r, pl.ds(g * 16, 16)] = i16 + (r * 128 + g * 16)

    @pl.when(s == 0)
    def _():
        for acc, nrow in ((a_cu, RU), (a_cs, RS), (a_du, RU), (a_ds, RS)):
            @pl.loop(0, nrow // 128)
            def _(k):
                pltpu.sync_copy(stage, acc.at[pl.ds(k * 128, 128)])

    plsc.subcore_barrier()

    t = c * 16 + s

    def count_phase(idx_ref, hist, ep, wref=None):
        et = ep // 32

        @pl.loop(0, et // 1024)
        def _(b):
            base = t * et + b * 1024
            pltpu.sync_copy(idx_ref.at[pl.ds(base, 1024)], st1)
            if wref is not None:
                pltpu.sync_copy(wref.at[pl.ds(base, 1024)], wst)

            @pl.loop(0, 64)
            def _(j):
                iv = st1[pl.ds(j * 16, 16)]
                vals = o16 if wref is None else wst[pl.ds(j * 16, 16)]
                plsc.addupdate_scatter(hist, [iv], vals)

    def reduce_phase(hist, acc, nrow):
        @pl.loop(0, nrow // 128)
        def _(k):
            @pl.loop(0, 128)
            def _(r):
                stage[r, pl.ds(0, 16)] = hist[pl.ds((k * 128 + r) * 16, 16)]
            pltpu.sync_copy(stage, acc.at[ridx.at[k]], add=True)

    def zero_hist(hist, n):
        @pl.loop(0, n // 16)
        def _(i):
            hist[pl.ds(i * 16, 16)] = z16

    eta = E_US_P // 32

    @pl.loop(0, eta // 1024)
    def _(b):
        base = t * eta + b * 1024
        pltpu.sync_copy(u_ref.at[pl.ds(base, 1024)], st1)
        pltpu.sync_copy(s_ref.at[pl.ds(base, 1024)], st2)

        @pl.loop(0, 64)
        def _(j):
            plsc.addupdate_scatter(h1, [st1[pl.ds(j * 16, 16)]], o16)
            plsc.addupdate_scatter(h2, [st2[pl.ds(j * 16, 16)]], o16)

    reduce_phase(h1, a_cu, RU)
    reduce_phase(h2, a_cs, RS)

    zero_hist(h1, NU)
    count_phase(ud_ref, h1, E_UE_P)
    reduce_phase(h1, a_du, RU)

    zero_hist(h2, MS)
    count_phase(sd_ref, h2, E_SP_P, w_ref)
    reduce_phase(h2, a_ds, RS)

    plsc.subcore_barrier()

    ru = RU // 16
    rs = RS // 16
    pltpu.sync_copy(a_cu.at[pl.ds(s * ru, ru)], cu_ref.at[c, pl.ds(s * ru, ru)])
    pltpu.sync_copy(a_cs.at[pl.ds(s * rs, rs)], cs_ref.at[c, pl.ds(s * rs, rs)])
    pltpu.sync_copy(a_du.at[pl.ds(s * ru, ru)], du_ref.at[c, pl.ds(s * ru, ru)])
    pltpu.sync_copy(a_ds.at[pl.ds(s * rs, rs)], ds_ref.at[c, pl.ds(s * rs, rs)])


_hist_call = pl.kernel(
    _hist_body,
    out_type=(
        jax.ShapeDtypeStruct((2, RU, 16), jnp.float32),
        jax.ShapeDtypeStruct((2, RS, 16), jnp.float32),
        jax.ShapeDtypeStruct((2, RU, 16), jnp.float32),
        jax.ShapeDtypeStruct((2, RS, 16), jnp.float32),
    ),
    mesh=_MESH,
    compiler_params=pltpu.CompilerParams(needs_layout_passes=False, use_tc_tiling_on_sc=False),
    scratch_types=[
        pltpu.VMEM((NU,), jnp.float32),
        pltpu.VMEM((MS,), jnp.float32),
        pltpu.VMEM((1024,), jnp.int32),
        pltpu.VMEM((1024,), jnp.int32),
        pltpu.VMEM((1024,), jnp.float32),
        pltpu.VMEM((RS // 128, 128), jnp.int32),
        pltpu.VMEM((128, 16), jnp.float32),
        pltpu.VMEM_SHARED((RU, 16), jnp.float32),
        pltpu.VMEM_SHARED((RS, 16), jnp.float32),
        pltpu.VMEM_SHARED((RU, 16), jnp.float32),
        pltpu.VMEM_SHARED((RS, 16), jnp.float32),
    ],
)


def _make_spmv(weighted, EP, VSP, VDP):
    def body(*refs):
        if weighted:
            (src_ref, dst_ref, tbl_ref, w_ref, out_ref,
             acc, src_st, dst_st, w_st, rA, rB, zer, gsA, gsB) = refs
        else:
            (src_ref, dst_ref, tbl_ref, out_ref,
             acc, src_st, dst_st, w_st, rA, rB, zer, gsA, gsB) = refs
        c = lax.axis_index("c")
        s = lax.axis_index("s")
        z16 = jnp.zeros((16,), jnp.float32)
        rowsb = (rA, rB)
        gsem = (gsA, gsB)

        @pl.loop(0, 64)
        def _(r):
            zer[r, pl.ds(0, 16)] = z16
            zer[r, pl.ds(16, 16)] = z16

        R = VDP // 16

        @pl.loop(0, R // 64)
        def _(k):
            pltpu.async_copy(zer, acc.at[pl.ds(s * R + k * 64, 64)], gsA)

        @pl.loop(0, R // 64)
        def _(k):
            pltpu.make_async_copy(zer, acc.at[pl.ds(s * R, 64)], gsA).wait()

        plsc.subcore_barrier()

        ET = EP // 16
        NB = ET // 1024
        NB2 = NB // 2

        def stage(b, half):
            base = s * ET + b * 1024
            pltpu.sync_copy(src_ref.at[c, pl.ds(base, 1024)], src_st)
            pltpu.sync_copy(dst_ref.at[pl.ds(s * (ET // 512) + b * 2, 2)],
                            dst_st.at[pl.ds(half * 2, 2)])
            if weighted:
                pltpu.sync_copy(w_ref.at[pl.ds(base, 1024)], w_st)

        def gis(k, buf):
            return pltpu.async_copy(
                tbl_ref.at[src_st.at[pl.ds(k * 512, 512)]], rowsb[buf],
                gsem[buf])

        def scale(buf, k):
            if not weighted:
                return
            rcur = rowsb[buf]

            @pl.loop(0, 32)
            def _(g):
                wv = w_st[pl.ds(k * 512 + g * 16, 16)]
                for j in range(16):
                    e = g * 16 + j
                    wsc = wv[j]
                    rcur[e, pl.ds(0, 16)] = rcur[e, pl.ds(0, 16)] * wsc
                    rcur[e, pl.ds(16, 16)] = rcur[e, pl.ds(16, 16)] * wsc

        def scat(half, k, buf):
            pltpu.sync_copy(rowsb[buf], acc.at[dst_st.at[half * 2 + k]],
                            add=True)

        stage(0, 0)
        gd = [gis(0, 0), gis(1, 1)]

        @pl.loop(0, NB2)
        def _(b2):
            b0 = b2 * 2
            gd[0].wait()
            scale(0, 0)
            scat(0, 0, 0)
            gd[1].wait()
            scale(1, 1)
            stage(b0 + 1, 1)
            gis(0, 0)
            scat(0, 1, 1)
            gis(1, 1)
            gd[0].wait()
            scale(0, 0)
            scat(1, 0, 0)

            @pl.when(b2 < NB2 - 1)
            def _():
                gd[1].wait()
                scale(1, 1)
                stage(b0 + 2, 0)
                gis(0, 0)
                scat(1, 1, 1)
                gis(1, 1)

            @pl.when(b2 == NB2 - 1)
            def _():
                gd[1].wait()
                scale(1, 1)
                scat(1, 1, 1)

        plsc.subcore_barrier()

        @pl.loop(0, R // 128)
        def _(k):
            r0_ = s * R + k * 128
            pltpu.async_copy(acc.at[pl.ds(r0_, 128)],
                             out_ref.at[pl.ds(c * VDP + r0_, 128)], gsA)

        @pl.loop(0, R // 128)
        def _(k):
            pltpu.make_async_copy(acc.at[pl.ds(s * R, 128)],
                                  out_ref.at[pl.ds(c * VDP + s * R, 128)],
                                  gsA).wait()

    return pl.kernel(
        body,
        out_type=jax.ShapeDtypeStruct((2 * VDP, HH), jnp.float32),
        mesh=_MESH,
        compiler_params=pltpu.CompilerParams(use_tc_tiling_on_sc=False),
        scratch_types=[
            pltpu.VMEM_SHARED((VDP, HH), jnp.float32),
            pltpu.VMEM((1024,), jnp.int32),
            pltpu.VMEM((4, 512), jnp.int32),
            pltpu.VMEM((1024,), jnp.float32),
            pltpu.VMEM((512, HH), jnp.float32),
            pltpu.VMEM((512, HH), jnp.float32),
            pltpu.VMEM((64, HH), jnp.float32),
            pltpu.SemaphoreType.DMA,
            pltpu.SemaphoreType.DMA,
        ],
    )


def _pad1(x, n, val):
    return jnp.pad(x, (0, n - x.shape[0]), constant_values=val)


def _stack_halves(x, vp):
    xp = jnp.pad(x, ((0, vp - x.shape[0]), (0, 0)))
    return jnp.concatenate([xp[:, :HH], xp[:, HH:]], axis=0)


def _unstack(xs, vr, vp):
    return jnp.concatenate([xs[:vr], xs[vp:vp + vr]], axis=1)


_spmv_uu = _make_spmv(False, E_UE_P, NU, NU)
_spmv_ss = _make_spmv(True, E_SP_P, MS, MS)
_spmv_su = _make_spmv(False, E_US_P, MS, NU)
_spmv_us = _make_spmv(False, E_US_P, NU, MS)


def kernel(spot_emb, user_emb, user_spot, spot_edge_index, spot_edge_weight,
           user_edge_index):
    u = _pad1(user_spot[0], E_US_P, N_USER)
    sp = _pad1(user_spot[1], E_US_P, M_SPOT)
    ue_s = _pad1(user_edge_index[0], E_UE_P, N_USER)
    ue_d = _pad1(user_edge_index[1], E_UE_P, N_USER)
    se_s = _pad1(spot_edge_index[0], E_SP_P, M_SPOT)
    se_d = _pad1(spot_edge_index[1], E_SP_P, M_SPOT)
    w = _pad1(spot_edge_weight, E_SP_P, 0.0)

    cu2, cs2, du2, ds2 = _hist_call(u, sp, ue_d, se_d, w)
    cnt_u = (cu2[0] + cu2[1]).reshape(NU)
    cnt_s = (cs2[0] + cs2[1]).reshape(MS)
    deg_u = (du2[0] + du2[1]).reshape(NU)
    deg_s = (ds2[0] + ds2[1]).reshape(MS)
    uinv = jnp.where(cnt_u > 0, lax.rsqrt(jnp.maximum(cnt_u, 1e-30)), 0.0)
    sinv = jnp.where(cnt_s > 0, lax.rsqrt(jnp.maximum(cnt_s, 1e-30)), 0.0)
    dui = jnp.where(deg_u > 0, lax.rsqrt(jnp.maximum(deg_u, 1e-30)), 0.0)
    dsi = jnp.where(deg_s > 0, lax.rsqrt(jnp.maximum(deg_s, 1e-30)), 0.0)
    uinv2 = jnp.concatenate([uinv, uinv])[:, None]
    sinv2 = jnp.concatenate([sinv, sinv])[:, None]
    dui2 = jnp.concatenate([dui, dui])[:, None]
    dsi2 = jnp.concatenate([dsi, dsi])[:, None]

    u2d = u.reshape(E_US_P // 512, 512)
    s2d = sp.reshape(E_US_P // 512, 512)
    ued2d = ue_d.reshape(E_UE_P // 512, 512)
    sed2d = se_d.reshape(E_SP_P // 512, 512)

    user_x = _stack_halves(user_emb, NU)
    spot_x = _stack_halves(spot_emb, MS)

    ue_s2 = jnp.stack([ue_s, ue_s + NU])
    se_s2 = jnp.stack([se_s, se_s + MS])
    u_2 = jnp.stack([u, u + NU])
    sp_2 = jnp.stack([sp, sp + MS])

    cat = _spmv_uu(ue_s2, ued2d, user_x * dui2)
    user_x = user_x + cat * dui2
    cat = _spmv_ss(se_s2, sed2d, spot_x * dsi2, w)
    spot_x = spot_x + cat * dsi2

    u_out = user_x
    s_out = spot_x
    for _ in range(3):
        nu_ = uinv2 * _spmv_su(sp_2, u2d, spot_x * sinv2)
        ns_ = sinv2 * _spmv_us(u_2, s2d, user_x * uinv2)
        user_x = nu_
        spot_x = ns_
        u_out = u_out + user_x
        s_out = s_out + spot_x

    s_out = s_out * 0.25
    u_out = u_out * 0.25
    return _unstack(s_out, M_SPOT, MS), _unstack(u_out, N_USER, NU)
